# initial kernel scaffold (unmeasured)
import jax
import jax.numpy as jnp
from jax import lax
from jax.experimental import pallas as pl
from jax.experimental.pallas import tpu as pltpu


def kernel(
    x,
):
    def body(*refs):
        pass

    out_shape = jax.ShapeDtypeStruct(..., jnp.float32)
    return pl.pallas_call(body, out_shape=out_shape)(...)


# baseline (device time: 26246 ns/iter reference)
import jax
import jax.numpy as jnp
from jax import lax
from jax.experimental import pallas as pl
from jax.experimental.pallas import tpu as pltpu

N_DEV = 32
M_PER = 8


def kernel(x):
    m, n = x.shape
    assert m == N_DEV * M_PER

    def body(x_ref, o_ref, rs_buf, rs_send_sems, rs_recv_sems,
             ag_send_sems, ag_recv_sems):
        me = lax.axis_index("i")

        rs_rdmas = []
        for d in range(1, N_DEV):
            dst = lax.rem(me + d, N_DEV)
            rdma = pltpu.make_async_remote_copy(
                src_ref=x_ref.at[pl.ds(dst * M_PER, M_PER), :],
                dst_ref=rs_buf.at[d],
                send_sem=rs_send_sems.at[d],
                recv_sem=rs_recv_sems.at[d],
                device_id=(dst,),
                device_id_type=pl.DeviceIdType.MESH,
            )
            rdma.start()
            rs_rdmas.append(rdma)

        rs_buf[0, :, :] = x_ref[pl.ds(me * M_PER, M_PER), :]

        for rdma in rs_rdmas:
            rdma.wait_recv()
        reduced = jnp.sum(rs_buf[...], axis=0)

        o_ref[pl.ds(me * M_PER, M_PER), :] = reduced
        ag_rdmas = []
        for d in range(1, N_DEV):
            dst = lax.rem(me + d, N_DEV)
            rdma = pltpu.make_async_remote_copy(
                src_ref=o_ref.at[pl.ds(me * M_PER, M_PER), :],
                dst_ref=o_ref.at[pl.ds(me * M_PER, M_PER), :],
                send_sem=ag_send_sems.at[d],
                recv_sem=ag_recv_sems.at[d],
                device_id=(dst,),
                device_id_type=pl.DeviceIdType.MESH,
            )
            rdma.start()
            ag_rdmas.append(rdma)

        for rdma in ag_rdmas:
            rdma.wait_recv()

        for rdma in rs_rdmas:
            rdma.wait_send()
        for rdma in ag_rdmas:
            rdma.wait_send()

    return pl.pallas_call(
        body,
        out_shape=jax.ShapeDtypeStruct((m, n), x.dtype),
        in_specs=[pl.BlockSpec(memory_space=pltpu.VMEM)],
        out_specs=pl.BlockSpec(memory_space=pltpu.VMEM),
        scratch_shapes=[
            pltpu.VMEM((N_DEV, M_PER, n), x.dtype),
            pltpu.SemaphoreType.DMA((N_DEV,)),
            pltpu.SemaphoreType.DMA((N_DEV,)),
            pltpu.SemaphoreType.DMA((N_DEV,)),
            pltpu.SemaphoreType.DMA((N_DEV,)),
        ],
    )(x)


# device time: 22487 ns/iter; 1.1672x vs baseline; 1.1672x over previous
import os

import jax
import jax.numpy as jnp
from jax import lax
from jax.experimental import pallas as pl
from jax.experimental.pallas import tpu as pltpu

N_DEV = 32
M_PER = 8
K_PHASES = int(os.environ.get("K_PHASES", "3"))


def kernel(x):
    m, n = x.shape
    assert m == N_DEV * M_PER

    def body(x_ref, o_ref, rs_buf, rs_send_sems, rs_recv_sems,
             ag_send_sems, ag_recv_sems):
        me = lax.axis_index("i")

        def flood_barrier(sem):
            for d in range(1, N_DEV):
                pl.semaphore_signal(
                    sem, inc=1,
                    device_id=(lax.rem(me + d, N_DEV),),
                    device_id_type=pl.DeviceIdType.MESH,
                )
            pl.semaphore_wait(sem, N_DEV - 1)

        with jax.named_scope("barrier"):
            flood_barrier(pltpu.get_barrier_semaphore())

        if K_PHASES == 1:
            o_ref[...] = x_ref[...]
            flood_barrier(pltpu.get_barrier_semaphore())
            return

        with jax.named_scope("rs_send"):
            rs_rdmas = []
            for d in range(1, N_DEV):
                dst = lax.rem(me + d, N_DEV)
                rdma = pltpu.make_async_remote_copy(
                    src_ref=x_ref.at[pl.ds(dst * M_PER, M_PER), :],
                    dst_ref=rs_buf.at[d],
                    send_sem=rs_send_sems.at[d],
                    recv_sem=rs_recv_sems.at[d],
                    device_id=(dst,),
                    device_id_type=pl.DeviceIdType.MESH,
                )
                rdma.start()
                rs_rdmas.append(rdma)

            rs_buf[0, :, :] = x_ref[pl.ds(me * M_PER, M_PER), :]

        with jax.named_scope("rs_wait"):
            for rdma in rs_rdmas:
                rdma.wait_recv()
        with jax.named_scope("reduce"):
            reduced = jnp.sum(rs_buf[...], axis=0)
            o_ref[pl.ds(me * M_PER, M_PER), :] = reduced

        if K_PHASES == 2:
            for rdma in rs_rdmas:
                rdma.wait_send()
            flood_barrier(pltpu.get_barrier_semaphore())
            return

        with jax.named_scope("ag_send"):
            ag_rdmas = []
            for d in range(1, N_DEV):
                dst = lax.rem(me + d, N_DEV)
                rdma = pltpu.make_async_remote_copy(
                    src_ref=o_ref.at[pl.ds(me * M_PER, M_PER), :],
                    dst_ref=o_ref.at[pl.ds(me * M_PER, M_PER), :],
                    send_sem=ag_send_sems.at[d],
                    recv_sem=ag_recv_sems.at[d],
                    device_id=(dst,),
                    device_id_type=pl.DeviceIdType.MESH,
                )
                rdma.start()
                ag_rdmas.append(rdma)

        with jax.named_scope("ag_wait"):
            for rdma in ag_rdmas:
                rdma.wait_recv()

            for rdma in rs_rdmas:
                rdma.wait_send()
            for rdma in ag_rdmas:
                rdma.wait_send()

    return pl.pallas_call(
        body,
        out_shape=jax.ShapeDtypeStruct((m, n), x.dtype),
        in_specs=[pl.BlockSpec(memory_space=pltpu.VMEM)],
        out_specs=pl.BlockSpec(memory_space=pltpu.VMEM),
        scratch_shapes=[
            pltpu.VMEM((N_DEV, M_PER, n), x.dtype),
            pltpu.SemaphoreType.DMA((N_DEV,)),
            pltpu.SemaphoreType.DMA((N_DEV,)),
            pltpu.SemaphoreType.DMA((N_DEV,)),
            pltpu.SemaphoreType.DMA((N_DEV,)),
        ],
        compiler_params=pltpu.CompilerParams(collective_id=0),
    )(x)


# device time: 18528 ns/iter; 1.4166x vs baseline; 1.2137x over previous
import jax
import jax.numpy as jnp
from jax import lax
from jax.experimental import pallas as pl
from jax.experimental.pallas import tpu as pltpu

N_DEV = 32
M_PER = 8

_FAR_FIRST = sorted(range(1, N_DEV), key=lambda d: -min(d, N_DEV - d))


def kernel(x):
    m, n = x.shape
    assert m == N_DEV * M_PER

    def body(x_ref, o_ref, rs_buf, rs_send_sems, rs_recv_sems,
             ag_send_sems, ag_recv_sems):
        me = lax.axis_index("i")

        with jax.named_scope("barrier"):
            barrier_sem = pltpu.get_barrier_semaphore()
            for d in _FAR_FIRST:
                pl.semaphore_signal(
                    barrier_sem, inc=1,
                    device_id=(lax.rem(me + d, N_DEV),),
                    device_id_type=pl.DeviceIdType.MESH,
                )
            pl.semaphore_wait(barrier_sem, N_DEV - 1)

        with jax.named_scope("rs_send"):
            rs_rdmas = []
            for d in _FAR_FIRST:
                dst = lax.rem(me + d, N_DEV)
                rdma = pltpu.make_async_remote_copy(
                    src_ref=x_ref.at[pl.ds(dst * M_PER, M_PER), :],
                    dst_ref=rs_buf.at[d - 1],
                    send_sem=rs_send_sems.at[d],
                    recv_sem=rs_recv_sems.at[d],
                    device_id=(dst,),
                    device_id_type=pl.DeviceIdType.MESH,
                )
                rdma.start()
                rs_rdmas.append(rdma)

        with jax.named_scope("rs_wait"):
            for rdma in rs_rdmas:
                rdma.wait_recv()
        with jax.named_scope("reduce"):
            reduced = (
                jnp.sum(rs_buf[...], axis=0)
                + x_ref[pl.ds(me * M_PER, M_PER), :]
            )
            o_ref[pl.ds(me * M_PER, M_PER), :] = reduced

        with jax.named_scope("ag_send"):
            ag_rdmas = []
            for d in _FAR_FIRST:
                dst = lax.rem(me + d, N_DEV)
                rdma = pltpu.make_async_remote_copy(
                    src_ref=o_ref.at[pl.ds(me * M_PER, M_PER), :],
                    dst_ref=o_ref.at[pl.ds(me * M_PER, M_PER), :],
                    send_sem=ag_send_sems.at[d],
                    recv_sem=ag_recv_sems.at[d],
                    device_id=(dst,),
                    device_id_type=pl.DeviceIdType.MESH,
                )
                rdma.start()
                ag_rdmas.append(rdma)

        with jax.named_scope("ag_wait"):
            for rdma in ag_rdmas:
                rdma.wait_recv()

            for rdma in rs_rdmas:
                rdma.wait_send()
            for rdma in ag_rdmas:
                rdma.wait_send()

    return pl.pallas_call(
        body,
        out_shape=jax.ShapeDtypeStruct((m, n), x.dtype),
        in_specs=[pl.BlockSpec(memory_space=pltpu.VMEM)],
        out_specs=pl.BlockSpec(memory_space=pltpu.VMEM),
        scratch_shapes=[
            pltpu.VMEM((N_DEV - 1, M_PER, n), x.dtype),
            pltpu.SemaphoreType.DMA((N_DEV,)),
            pltpu.SemaphoreType.DMA((N_DEV,)),
            pltpu.SemaphoreType.DMA((N_DEV,)),
            pltpu.SemaphoreType.DMA((N_DEV,)),
        ],
        compiler_params=pltpu.CompilerParams(collective_id=0),
    )(x)
